# Initial kernel scaffold; baseline (speedup 1.0000x reference)
import jax
import jax.numpy as jnp
from jax.experimental import pallas as pl


def kernel(pred, target):
    raise NotImplementedError("write your pallas kernel here")


# speedup vs baseline: 13.7738x; 13.7738x over previous
import jax
import jax.numpy as jnp
from jax.experimental import pallas as pl

_B, _N, _D = 1024, 128, 128
_BB = 64


def _lap_l1_kernel(p_ref, t_ref, out_ref):
    i = pl.program_id(0)
    d = p_ref[...] - t_ref[...]
    up = jnp.roll(d, 1, axis=1)
    dn = jnp.roll(d, -1, axis=1)
    lap = d - 0.5 * (up + dn)
    part = jnp.sum(jnp.abs(lap)).reshape(1, 1)

    @pl.when(i == 0)
    def _():
        out_ref[...] = jnp.zeros_like(out_ref)

    out_ref[...] += part


def kernel(pred, target):
    s = pl.pallas_call(
        _lap_l1_kernel,
        grid=(_B // _BB,),
        in_specs=[
            pl.BlockSpec((_BB, _N, _D), lambda i: (i, 0, 0)),
            pl.BlockSpec((_BB, _N, _D), lambda i: (i, 0, 0)),
        ],
        out_specs=pl.BlockSpec((1, 1), lambda i: (0, 0)),
        out_shape=jax.ShapeDtypeStruct((1, 1), jnp.float32),
    )(pred, target)
    return s[0, 0] / (_B * _N * _D)
